# baseline (speedup 1.0000x reference)
import functools
import jax
import jax.numpy as jnp
from jax import lax
from jax.experimental import pallas as pl
from jax.experimental.pallas import tpu as pltpu
from jax.experimental.pallas import tpu_sc as plsc

B = 16384
D = 64
V = 1000000
L = 16
CHUNK = 128
GPC = CHUNK // L
EPC = CHUNK * D

_info = plsc.get_sparse_core_info()
NC = _info.num_cores
NS = _info.num_subcores
NW = NC * NS
B_PER_W = B // NW
N_CHUNKS = B_PER_W // CHUNK


def _sc_kernel(center_hbm, context_hbm, fa, fb, dot_hbm,
               idx_c, idx_x, ea, eb, ga, gb, out_v, sem):
    wid = lax.axis_index("s") * NC + lax.axis_index("c")
    base = wid * B_PER_W

    pltpu.sync_copy(center_hbm.at[pl.ds(base, B_PER_W)], idx_c)
    pltpu.sync_copy(context_hbm.at[pl.ds(base, B_PER_W)], idx_x)

    def chunk_body(ci, _):
        def build(g, _):
            va = idx_c[pl.ds(ci * CHUNK + g * L, L)]
            vx = idx_x[pl.ds(ci * CHUNK + g * L, L)]
            for d in range(D):
                ea[pl.ds((g * D + d) * L, L)] = va + d * V
                eb[pl.ds((g * D + d) * L, L)] = vx + d * V
            return 0

        lax.fori_loop(0, GPC, build, 0)

        copies = []
        for j in range(EPC // CHUNK):
            s = pl.ds(j * CHUNK, CHUNK)
            copies.append(pltpu.async_copy(fa.at[ea.at[s]], ga.at[s], sem))
            copies.append(pltpu.async_copy(fb.at[eb.at[s]], gb.at[s], sem))
        for cp in copies:
            cp.wait()

        def dot(g, _):
            acc = None
            for d in range(D):
                s = pl.ds((g * D + d) * L, L)
                prod = ga[s] * gb[s]
                acc = prod if acc is None else acc + prod
            out_v[pl.ds(ci * CHUNK + g * L, L)] = acc
            return 0

        lax.fori_loop(0, GPC, dot, 0)
        return 0

    lax.fori_loop(0, N_CHUNKS, chunk_body, 0)

    pltpu.sync_copy(out_v, dot_hbm.at[pl.ds(base, B_PER_W)])


@jax.jit
def _word2vec_dot(center, context, fa, fb):
    mesh = plsc.VectorSubcoreMesh(core_axis_name="c", subcore_axis_name="s")
    k = functools.partial(
        pl.kernel,
        out_type=jax.ShapeDtypeStruct((B,), jnp.float32),
        mesh=mesh,
        compiler_params=pltpu.CompilerParams(
            needs_layout_passes=False, use_tc_tiling_on_sc=False),
        scratch_types=[
            pltpu.VMEM((B_PER_W,), jnp.int32),
            pltpu.VMEM((B_PER_W,), jnp.int32),
            pltpu.VMEM((EPC,), jnp.int32),
            pltpu.VMEM((EPC,), jnp.int32),
            pltpu.VMEM((EPC,), jnp.float32),
            pltpu.VMEM((EPC,), jnp.float32),
            pltpu.VMEM((B_PER_W,), jnp.float32),
            pltpu.SemaphoreType.DMA,
        ],
    )(_sc_kernel)
    return k(center, context, fa, fb)


def kernel(center, context, in_table, out_table):
    fa = in_table.T.reshape(-1)
    fb = out_table.T.reshape(-1)
    return _word2vec_dot(center.astype(jnp.int32), context.astype(jnp.int32),
                         fa, fb)
